# baseline (speedup 1.0000x reference)
import math
import functools

import jax
import jax.numpy as jnp
from jax import lax
from jax.experimental import pallas as pl
from jax.experimental.pallas import tpu as pltpu
from jax.experimental.pallas import tpu_sc as plsc

D_MODEL = 1024
MAX_LEN = 8192
BATCH = 4
LANES = 16
NUM_CORES = 2
NUM_SUBCORES = 16
NUM_WORKERS = NUM_CORES * NUM_SUBCORES
SEQ_PER_WORKER = MAX_LEN // NUM_WORKERS
CHUNK = 32


def _make_pe(max_len, d_model):
    pos = jnp.arange(max_len, dtype=jnp.float32)[:, None]
    div = jnp.exp(jnp.arange(0, d_model, 2, dtype=jnp.float32)
                  * (-math.log(10000.0) / d_model))
    pe = jnp.zeros((max_len, d_model), dtype=jnp.float32)
    pe = pe.at[:, 0::2].set(jnp.sin(pos * div))
    pe = pe.at[:, 1::2].set(jnp.cos(pos * div))
    return pe


_mesh = plsc.VectorSubcoreMesh(
    core_axis_name="c", subcore_axis_name="s",
    num_cores=NUM_CORES, num_subcores=NUM_SUBCORES)


@functools.partial(
    pl.kernel,
    out_type=jax.ShapeDtypeStruct((BATCH * MAX_LEN, D_MODEL), jnp.float32),
    mesh=_mesh,
    scratch_types=[
        pltpu.VMEM((CHUNK,), jnp.int32),
        pltpu.VMEM((CHUNK, D_MODEL), jnp.float32),
        pltpu.VMEM((CHUNK, D_MODEL), jnp.float32),
        pltpu.SemaphoreType.DMA,
    ],
)
def _sc_embed(x_hbm, table_hbm, pe_hbm, out_hbm, idx_v, pe_v, rows_v, sem):
    wid = lax.axis_index("s") * NUM_CORES + lax.axis_index("c")
    s_base = wid * SEQ_PER_WORKER

    def chunk_step(j, carry):
        s0 = s_base + j * CHUNK
        pltpu.sync_copy(pe_hbm.at[pl.ds(s0, CHUNK)], pe_v)

        def batch_step(b, carry2):
            off = b * MAX_LEN + s0
            pltpu.sync_copy(x_hbm.at[pl.ds(off, CHUNK)], idx_v)
            pltpu.async_copy(table_hbm.at[idx_v], rows_v, sem).wait()

            def row_add(r, carry3):
                for k in range(D_MODEL // LANES):
                    sl = pl.ds(k * LANES, LANES)
                    rows_v[r, sl] = rows_v[r, sl] + pe_v[r, sl]
                return carry3

            lax.fori_loop(0, CHUNK, row_add, 0, unroll=False)
            pltpu.sync_copy(rows_v, out_hbm.at[pl.ds(off, CHUNK)])
            return carry2

        lax.fori_loop(0, BATCH, batch_step, 0, unroll=False)
        return carry

    lax.fori_loop(0, SEQ_PER_WORKER // CHUNK, chunk_step, 0, unroll=False)


def kernel(x, table):
    pe = _make_pe(MAX_LEN, D_MODEL)
    idx = x.reshape(BATCH * MAX_LEN).astype(jnp.int32)
    out = _sc_embed(idx, table, pe)
    return out.reshape(BATCH, MAX_LEN, D_MODEL)
